# baseline (speedup 1.0000x reference)
import functools

import jax
import jax.numpy as jnp
from jax import lax
from jax.experimental import pallas as pl
from jax.experimental.pallas import tpu as pltpu
from jax.experimental.pallas import tpu_sc as plsc

_COARSE = 0.3
_MEDIUM = 0.4
_N16 = 256 * 32 * 32
_N8 = 256 * 64 * 64
_K1 = round(_N16 * _COARSE)
_K2 = round(4 * _N16 * _COARSE + _N8 * _MEDIUM)


def _count_less(arrs, t):
    s = jnp.int32(0)
    for a in arrs:
        s = s + jnp.sum((a < t).astype(jnp.int32))
    return s


def _bisect_kth(arrs, k):
    lo = jnp.int32(0)
    for r in range(15):
        w = 1 << (28 - 2 * r)
        t1 = lo + w
        t2 = lo + 2 * w
        t3 = lo + 3 * w
        s1 = _count_less(arrs, t1)
        s2 = _count_less(arrs, t2)
        s3 = _count_less(arrs, t3)
        lo = jnp.where(k <= s1, lo,
                       jnp.where(k <= s2, t1,
                                 jnp.where(k <= s3, t2, t3)))
    return lo


def _select_body(x16_ref, x8_ref, oc_ref, om_ref):
    v16 = lax.bitcast_convert_type(x16_ref[...], jnp.int32)
    c_bits = _bisect_kth([v16], _K1)

    m16 = (v16 < c_bits).astype(jnp.bfloat16)

    x8v = x8_ref[...]
    row = lax.broadcasted_iota(jnp.int32, (128, 128), 0)
    col = lax.broadcasted_iota(jnp.int32, (128, 128), 1)
    masked = []
    for r in range(4):
        p_r = (row == (32 * r + (col % 64) // 2)).astype(jnp.bfloat16)
        mcols = jnp.dot(m16, p_r, preferred_element_type=jnp.float32)
        vals = jnp.where(mcols > 0.5, 0.0, x8v[:, 128 * r:128 * (r + 1)])
        masked.append(lax.bitcast_convert_type(vals, jnp.int32))
    m_bits = _bisect_kth(masked, _K2)

    oc_ref[0, 0] = c_bits
    om_ref[0, 0] = m_bits


def _gates_body(gct_ref, gmt_ref, gf_ref, gl_ref):
    gc4p = jnp.repeat(gct_ref[0], 4, axis=0)
    gm0 = jnp.repeat(gmt_ref[0], 2, axis=0)
    gm1 = jnp.repeat(gmt_ref[1], 2, axis=0)
    for e in range(4):
        gme = gm0 if e < 2 else gm1
        gfe = 1 - gc4p - gme
        gf_ref[e] = gfe
        gl_ref[0, e] = gc4p
        gl_ref[1, e] = gme
        gl_ref[2, e] = gfe


def _sc_small_gates(x16t, x8t, ct16, mt16):
    mesh = plsc.VectorSubcoreMesh(core_axis_name="c", subcore_axis_name="s")

    @functools.partial(
        pl.kernel,
        mesh=mesh,
        out_type=(jax.ShapeDtypeStruct((262144,), jnp.int32),
                  jax.ShapeDtypeStruct((1048576,), jnp.int32)),
        scratch_types=[pltpu.VMEM((16,), jnp.float32),
                       pltpu.VMEM((16,), jnp.float32),
                       pltpu.VMEM((8192,), jnp.float32),
                       pltpu.VMEM((32768,), jnp.float32),
                       pltpu.VMEM((8192,), jnp.int32),
                       pltpu.VMEM((32768,), jnp.int32)],
    )
    def run(x16_hbm, x8_hbm, ct_hbm, mt_hbm, gct_hbm, gmt_hbm,
            ct_v, mt_v, x16_v, x8_v, gct_v, gmt_v):
        w = lax.axis_index("s") * 2 + lax.axis_index("c")
        b16 = w * 8192
        b8 = w * 32768
        pltpu.sync_copy(ct_hbm, ct_v)
        pltpu.sync_copy(mt_hbm, mt_v)
        pltpu.sync_copy(x16_hbm.at[pl.ds(b16, 8192)], x16_v)
        pltpu.sync_copy(x8_hbm.at[pl.ds(b8, 32768)], x8_v)
        ctv = ct_v[...]
        mtv = mt_v[...]

        def body16(i, carry):
            v = x16_v[pl.ds(i * 16, 16)]
            gct_v[pl.ds(i * 16, 16)] = jnp.where(v < ctv, 1, 0)
            return carry

        lax.fori_loop(0, 512, body16, 0, unroll=8)

        def body8(i, carry):
            p0 = i * 16
            j8 = (p0 // 256) % 64
            par = gct_v[pl.ds((j8 // 2) * 256 + p0 % 256, 16)]
            v = x8_v[pl.ds(p0, 16)]
            gmt_v[pl.ds(p0, 16)] = jnp.where((v < mtv) & (par == 0), 1, 0)
            return carry

        lax.fori_loop(0, 2048, body8, 0, unroll=8)
        pltpu.sync_copy(gct_v, gct_hbm.at[pl.ds(b16, 8192)])
        pltpu.sync_copy(gmt_v, gmt_hbm.at[pl.ds(b8, 32768)])

    return run(x16t, x8t, ct16, mt16)


@jax.jit
def kernel(x_entropy_p16, x_entropy_p8):
    x16f = x_entropy_p16.reshape(2048, 128)
    x8f = x_entropy_p8.reshape(2048, 512)

    c_bits, m_bits = pl.pallas_call(
        _select_body,
        out_specs=(pl.BlockSpec(memory_space=pltpu.SMEM),
                   pl.BlockSpec(memory_space=pltpu.SMEM)),
        out_shape=(jax.ShapeDtypeStruct((1, 1), jnp.int32),
                   jax.ShapeDtypeStruct((1, 1), jnp.int32)),
    )(x16f, x8f)
    cthr = lax.bitcast_convert_type(c_bits, jnp.float32)
    mthr = lax.bitcast_convert_type(m_bits, jnp.float32)

    ct16 = jnp.broadcast_to(cthr.reshape(1), (16,))
    mt16 = jnp.broadcast_to(mthr.reshape(1), (16,))
    gct, gmt = _sc_small_gates(
        x_entropy_p16.transpose(1, 2, 0).reshape(-1),
        x_entropy_p8.transpose(1, 2, 0).reshape(-1), ct16, mt16)
    gct = gct.reshape(32, 32, 256)
    gmt = gmt.reshape(64, 64, 256)

    gf, gl = pl.pallas_call(
        _gates_body,
        grid=(32,),
        in_specs=[
            pl.BlockSpec((1, 32, 256), lambda q: (q, 0, 0)),
            pl.BlockSpec((2, 64, 256), lambda q: (q, 0, 0)),
        ],
        out_specs=[
            pl.BlockSpec((4, 128, 256), lambda q: (q, 0, 0)),
            pl.BlockSpec((3, 4, 128, 256), lambda q: (0, q, 0, 0)),
        ],
        out_shape=(
            jax.ShapeDtypeStruct((128, 128, 256), jnp.int32),
            jax.ShapeDtypeStruct((3, 128, 128, 256), jnp.int32),
        ),
    )(gct, gmt)
    return (gct.transpose(2, 0, 1), gmt.transpose(2, 0, 1),
            gf.transpose(2, 0, 1), gl.transpose(3, 1, 2, 0))


# speedup vs baseline: 1.4773x; 1.4773x over previous
import functools

import jax
import jax.numpy as jnp
from jax import lax
from jax.experimental import pallas as pl
from jax.experimental.pallas import tpu as pltpu
from jax.experimental.pallas import tpu_sc as plsc

_COARSE = 0.3
_MEDIUM = 0.4
_N16 = 256 * 32 * 32
_N8 = 256 * 64 * 64
_K1 = round(_N16 * _COARSE)
_K2 = round(4 * _N16 * _COARSE + _N8 * _MEDIUM)


def _count_less(arrs, t):
    s = jnp.int32(0)
    for a in arrs:
        s = s + jnp.sum((a < t).astype(jnp.int32))
    return s


def _bisect_kth(arrs, k):
    lo = jnp.int32(0)
    for r in range(15):
        w = 1 << (28 - 2 * r)
        t1 = lo + w
        t2 = lo + 2 * w
        t3 = lo + 3 * w
        s1 = _count_less(arrs, t1)
        s2 = _count_less(arrs, t2)
        s3 = _count_less(arrs, t3)
        lo = jnp.where(k <= s1, lo,
                       jnp.where(k <= s2, t1,
                                 jnp.where(k <= s3, t2, t3)))
    return lo


def _select_body(x16_ref, x8_ref, oc_ref, om_ref):
    v16 = lax.bitcast_convert_type(x16_ref[...], jnp.int32)
    c_bits = _bisect_kth([v16], _K1)

    m16 = (v16 < c_bits).astype(jnp.bfloat16)

    x8v = x8_ref[...]
    row = lax.broadcasted_iota(jnp.int32, (128, 128), 0)
    col = lax.broadcasted_iota(jnp.int32, (128, 128), 1)
    masked = []
    for r in range(4):
        p_r = (row == (32 * r + (col % 64) // 2)).astype(jnp.bfloat16)
        mcols = jnp.dot(m16, p_r, preferred_element_type=jnp.float32)
        vals = jnp.where(mcols > 0.5, 0.0, x8v[:, 128 * r:128 * (r + 1)])
        masked.append(lax.bitcast_convert_type(vals, jnp.int32))
    m_bits = _bisect_kth(masked, _K2)

    oc_ref[0, 0] = c_bits
    om_ref[0, 0] = m_bits


def _expand(rows, cols):
    return (rows[:, None] == cols[None, :]).astype(jnp.bfloat16)


def _gates_body(x16_ref, x8_ref, ct_ref, mt_ref, gf_ref, gl_ref):
    cthr = ct_ref[0, 0]
    mthr = mt_ref[0, 0]
    x16 = x16_ref[...].reshape(256, 32)
    x8 = x8_ref[...].reshape(512, 64)

    i128 = lax.iota(jnp.int32, 128)
    i64 = lax.iota(jnp.int32, 64)
    i32 = lax.iota(jnp.int32, 32)

    gcf = (x16 < cthr).astype(jnp.bfloat16)

    c2c = _expand(i32, i64 // 2)
    gc2 = jnp.dot(jnp.repeat(gcf, 2, axis=0), c2c,
                  preferred_element_type=jnp.float32)

    gmf = ((x8 < mthr) & (gc2 < 0.5)).astype(jnp.bfloat16)

    c4c = _expand(i32, i128 // 4)
    gc4 = jnp.dot(jnp.repeat(gcf, 4, axis=0), c4c,
                  preferred_element_type=jnp.float32)

    c2f = _expand(i64, i128 // 2)
    gm2 = jnp.dot(jnp.repeat(gmf, 2, axis=0), c2f,
                  preferred_element_type=jnp.float32)

    gff = 1.0 - gc4 - gm2
    gf_ref[...] = gff.astype(jnp.int32).reshape(8, 128, 128)

    gl_ref[:, 0] = gc4.astype(jnp.int32).reshape(8, 128, 128)
    gl_ref[:, 1] = gm2.astype(jnp.int32).reshape(8, 128, 128)
    gl_ref[:, 2] = gff.astype(jnp.int32).reshape(8, 128, 128)


def _sc_small_gates(x16t, x8t, ct16, mt16):
    mesh = plsc.VectorSubcoreMesh(core_axis_name="c", subcore_axis_name="s")

    @functools.partial(
        pl.kernel,
        mesh=mesh,
        out_type=(jax.ShapeDtypeStruct((262144,), jnp.int32),
                  jax.ShapeDtypeStruct((1048576,), jnp.int32)),
        scratch_types=[pltpu.VMEM((16,), jnp.float32),
                       pltpu.VMEM((16,), jnp.float32),
                       pltpu.VMEM((8192,), jnp.float32),
                       pltpu.VMEM((32768,), jnp.float32),
                       pltpu.VMEM((8192,), jnp.int32),
                       pltpu.VMEM((32768,), jnp.int32)],
    )
    def run(x16_hbm, x8_hbm, ct_hbm, mt_hbm, gct_hbm, gmt_hbm,
            ct_v, mt_v, x16_v, x8_v, gct_v, gmt_v):
        w = lax.axis_index("s") * 2 + lax.axis_index("c")
        b16 = w * 8192
        b8 = w * 32768
        pltpu.sync_copy(ct_hbm, ct_v)
        pltpu.sync_copy(mt_hbm, mt_v)
        pltpu.sync_copy(x16_hbm.at[pl.ds(b16, 8192)], x16_v)
        pltpu.sync_copy(x8_hbm.at[pl.ds(b8, 32768)], x8_v)
        ctv = ct_v[...]
        mtv = mt_v[...]

        def body16(i, carry):
            v = x16_v[pl.ds(i * 16, 16)]
            gct_v[pl.ds(i * 16, 16)] = jnp.where(v < ctv, 1, 0)
            return carry

        lax.fori_loop(0, 512, body16, 0, unroll=8)

        def body8(i, carry):
            p0 = i * 16
            j8 = (p0 // 256) % 64
            par = gct_v[pl.ds((j8 // 2) * 256 + p0 % 256, 16)]
            v = x8_v[pl.ds(p0, 16)]
            gmt_v[pl.ds(p0, 16)] = jnp.where((v < mtv) & (par == 0), 1, 0)
            return carry

        lax.fori_loop(0, 2048, body8, 0, unroll=8)
        pltpu.sync_copy(gct_v, gct_hbm.at[pl.ds(b16, 8192)])
        pltpu.sync_copy(gmt_v, gmt_hbm.at[pl.ds(b8, 32768)])

    return run(x16t, x8t, ct16, mt16)


@jax.jit
def kernel(x_entropy_p16, x_entropy_p8):
    x16f = x_entropy_p16.reshape(2048, 128)
    x8f = x_entropy_p8.reshape(2048, 512)

    c_bits, m_bits = pl.pallas_call(
        _select_body,
        out_specs=(pl.BlockSpec(memory_space=pltpu.SMEM),
                   pl.BlockSpec(memory_space=pltpu.SMEM)),
        out_shape=(jax.ShapeDtypeStruct((1, 1), jnp.int32),
                   jax.ShapeDtypeStruct((1, 1), jnp.int32)),
    )(x16f, x8f)
    cthr = lax.bitcast_convert_type(c_bits, jnp.float32)
    mthr = lax.bitcast_convert_type(m_bits, jnp.float32)

    ct16 = jnp.broadcast_to(cthr.reshape(1), (16,))
    mt16 = jnp.broadcast_to(mthr.reshape(1), (16,))
    gct, gmt = _sc_small_gates(
        x_entropy_p16.transpose(1, 2, 0).reshape(-1),
        x_entropy_p8.transpose(1, 2, 0).reshape(-1), ct16, mt16)
    gct = gct.reshape(32, 32, 256)
    gmt = gmt.reshape(64, 64, 256)

    gf, gl = pl.pallas_call(
        _gates_body,
        grid=(32,),
        in_specs=[
            pl.BlockSpec((8, 32, 32), lambda b: (b, 0, 0)),
            pl.BlockSpec((8, 64, 64), lambda b: (b, 0, 0)),
            pl.BlockSpec((1, 1), lambda b: (0, 0)),
            pl.BlockSpec((1, 1), lambda b: (0, 0)),
        ],
        out_specs=[
            pl.BlockSpec((8, 128, 128), lambda b: (b, 0, 0)),
            pl.BlockSpec((8, 3, 128, 128), lambda b: (b, 0, 0, 0)),
        ],
        out_shape=(
            jax.ShapeDtypeStruct((256, 128, 128), jnp.int32),
            jax.ShapeDtypeStruct((256, 3, 128, 128), jnp.int32),
        ),
    )(x_entropy_p16, x_entropy_p8, cthr, mthr)
    return (gct.transpose(2, 0, 1), gmt.transpose(2, 0, 1), gf,
            gl.transpose(0, 2, 3, 1))
